# Initial kernel scaffold; baseline (speedup 1.0000x reference)
import jax
import jax.numpy as jnp
from jax.experimental import pallas as pl


def kernel(x, e_feat, e_attr, emb_table):
    raise NotImplementedError("write your pallas kernel here")


# speedup vs baseline: 3.5615x; 3.5615x over previous
import functools

import jax
import jax.numpy as jnp
from jax import lax
from jax.experimental import pallas as pl
from jax.experimental.pallas import tpu as pltpu
from jax.experimental.pallas import tpu_sc as plsc

N = 10000
D = 128
E = 320000
NC = 2
NS = 16
K = 128
CHUNKS = 81
EPS = K * CHUNKS
EPC = EPS * NS
EP = EPC * NC
OUT_ROWS = 10240
ZROWS = 64
WB = OUT_ROWS // NS


def _sc_kernel(x, pair, src, dst, code, out, out_sh, idx_v, xbuf, ebuf,
               zbuf):
    c = lax.axis_index("c")
    s = lax.axis_index("s")

    z = jnp.zeros((16,), jnp.float32)

    def zero_row(i, carry):
        for k in range(D // 16):
            zbuf[i, pl.ds(k * 16, 16)] = z
        return carry

    lax.fori_loop(0, ZROWS, zero_row, 0)

    slab = s * WB

    def zero_slab(j, carry):
        pltpu.sync_copy(zbuf, out_sh.at[pl.ds(slab + j * ZROWS, ZROWS)])
        return carry

    lax.fori_loop(0, WB // ZROWS, zero_slab, 0)

    plsc.subcore_barrier()

    def chunk(g, carry):
        base = c * EPC + s * EPS + g * K
        pltpu.sync_copy(src.at[pl.ds(base, K)], idx_v.at[0])
        pltpu.sync_copy(dst.at[pl.ds(base, K)], idx_v.at[1])
        pltpu.sync_copy(code.at[pl.ds(base, K)], idx_v.at[2])
        pltpu.sync_copy(x.at[idx_v.at[0]], xbuf)
        pltpu.sync_copy(pair.at[idx_v.at[2]], ebuf)
        pltpu.sync_copy(xbuf, out_sh.at[idx_v.at[1]], add=True)
        pltpu.sync_copy(ebuf, out_sh.at[idx_v.at[1]], add=True)
        return carry

    lax.fori_loop(0, CHUNKS, chunk, 0)

    plsc.subcore_barrier()

    pltpu.sync_copy(out_sh.at[pl.ds(slab, WB)], out.at[c].at[pl.ds(slab, WB)])


def _merge_kernel(p_ref, o_ref):
    o_ref[...] = p_ref[0] + p_ref[1]


@jax.jit
def _propagate(x, pair, src, dst, code):
    mesh = plsc.VectorSubcoreMesh(core_axis_name="c", subcore_axis_name="s")
    partials = pl.kernel(
        _sc_kernel,
        out_type=jax.ShapeDtypeStruct((NC, OUT_ROWS, D), jnp.float32),
        mesh=mesh,
        scratch_types=[
            pltpu.VMEM_SHARED((OUT_ROWS, D), jnp.float32),
            pltpu.VMEM((4, K), jnp.int32),
            pltpu.VMEM((K, D), jnp.float32),
            pltpu.VMEM((K, D), jnp.float32),
            pltpu.VMEM((ZROWS, D), jnp.float32),
        ],
    )(x, pair, src, dst, code)
    blk = 1024
    merged = pl.pallas_call(
        _merge_kernel,
        grid=(OUT_ROWS // blk,),
        in_specs=[pl.BlockSpec((NC, blk, D), lambda i: (0, i, 0))],
        out_specs=pl.BlockSpec((blk, D), lambda i: (i, 0)),
        out_shape=jax.ShapeDtypeStruct((OUT_ROWS, D), jnp.float32),
    )(partials)
    return merged


def kernel(x, e_feat, e_attr, emb_table):
    src = e_feat[0].astype(jnp.int32)
    dst = e_feat[1].astype(jnp.int32)
    code = (e_attr[:, 0] * 32 + e_attr[:, 1]).astype(jnp.int32)
    loop = jnp.arange(N, dtype=jnp.int32)
    pad = EP - E - N
    srcp = jnp.concatenate([src, loop, jnp.zeros((pad,), jnp.int32)])
    dstp = jnp.concatenate([dst, loop, jnp.full((pad,), N, jnp.int32)])
    codep = jnp.concatenate(
        [code, jnp.full((N,), 4 * 32, jnp.int32), jnp.zeros((pad,), jnp.int32)]
    )
    pair = (emb_table[:, None, :] + emb_table[None, :, :]).reshape(1024, D)
    out = _propagate(x, pair, srcp, dstp, codep)
    return out[:N]


# speedup vs baseline: 4.5976x; 1.2909x over previous
import functools

import jax
import jax.numpy as jnp
from jax import lax
from jax.experimental import pallas as pl
from jax.experimental.pallas import tpu as pltpu
from jax.experimental.pallas import tpu_sc as plsc

N = 10000
D = 128
E = 320000
NC = 2
NS = 16
K = 128
CHUNKS = 162
IPS = K * CHUNKS
IPC = IPS * NS
IP = IPC * NC
OUT_ROWS = 10240
ZROWS = 32
WB = OUT_ROWS // NS


def _sc_kernel(table, gl, dl, out, out_sh, ibuf0, ibuf1, xbuf0, xbuf1,
               zbuf, gsem0, gsem1, ssem0, ssem1):
    c = lax.axis_index("c")
    s = lax.axis_index("s")

    z = jnp.zeros((16,), jnp.float32)

    def zero_row(i, carry):
        for k in range(D // 16):
            zbuf[i, pl.ds(k * 16, 16)] = z
        return carry

    lax.fori_loop(0, ZROWS, zero_row, 0)

    slab = s * WB

    def zero_slab(j, carry):
        pltpu.sync_copy(zbuf, out_sh.at[pl.ds(slab + j * ZROWS, ZROWS)])
        return carry

    lax.fori_loop(0, WB // ZROWS, zero_slab, 0)

    plsc.subcore_barrier()

    first = c * IPC + s * IPS

    def load_idx(g, ib):
        base = first + g * K
        pltpu.sync_copy(gl.at[pl.ds(base, K)], ib.at[0])
        pltpu.sync_copy(dl.at[pl.ds(base, K)], ib.at[1])

    def gather(ib, xb, sem):
        pltpu.async_copy(table.at[ib.at[0]], xb, sem)

    def wait_gather(ib, xb, sem):
        pltpu.make_async_copy(table.at[ib.at[0]], xb, sem).wait()

    def scatter(ib, xb, sem):
        pltpu.async_copy(xb, out_sh.at[ib.at[1]], sem, add=True)

    def wait_scatter(ib, xb, sem):
        pltpu.make_async_copy(xb, out_sh.at[ib.at[1]], sem).wait()

    load_idx(0, ibuf0)
    gather(ibuf0, xbuf0, gsem0)

    def step(t, carry):
        g0 = 2 * t

        wait_gather(ibuf0, xbuf0, gsem0)
        scatter(ibuf0, xbuf0, ssem0)

        @pl.when(t > 0)
        def _():
            wait_scatter(ibuf1, xbuf1, ssem1)

        load_idx(g0 + 1, ibuf1)
        gather(ibuf1, xbuf1, gsem1)

        wait_scatter(ibuf0, xbuf0, ssem0)

        @pl.when(g0 + 2 < CHUNKS)
        def _():
            load_idx(g0 + 2, ibuf0)
            gather(ibuf0, xbuf0, gsem0)

        wait_gather(ibuf1, xbuf1, gsem1)
        scatter(ibuf1, xbuf1, ssem1)
        return carry

    lax.fori_loop(0, CHUNKS // 2, step, 0)
    wait_scatter(ibuf1, xbuf1, ssem1)

    plsc.subcore_barrier()

    pltpu.sync_copy(out_sh.at[pl.ds(slab, WB)], out.at[c].at[pl.ds(slab, WB)])


def _merge_kernel(p_ref, o_ref):
    o_ref[...] = p_ref[0] + p_ref[1]


@jax.jit
def _propagate(table, gl, dl):
    mesh = plsc.VectorSubcoreMesh(core_axis_name="c", subcore_axis_name="s")
    partials = pl.kernel(
        _sc_kernel,
        out_type=jax.ShapeDtypeStruct((NC, OUT_ROWS, D), jnp.float32),
        mesh=mesh,
        scratch_types=[
            pltpu.VMEM_SHARED((OUT_ROWS, D), jnp.float32),
            pltpu.VMEM((2, K), jnp.int32),
            pltpu.VMEM((2, K), jnp.int32),
            pltpu.VMEM((K, D), jnp.float32),
            pltpu.VMEM((K, D), jnp.float32),
            pltpu.VMEM((ZROWS, D), jnp.float32),
            pltpu.SemaphoreType.DMA,
            pltpu.SemaphoreType.DMA,
            pltpu.SemaphoreType.DMA,
            pltpu.SemaphoreType.DMA,
        ],
    )(table, gl, dl)
    blk = 1024
    merged = pl.pallas_call(
        _merge_kernel,
        grid=(OUT_ROWS // blk,),
        in_specs=[pl.BlockSpec((NC, blk, D), lambda i: (0, i, 0))],
        out_specs=pl.BlockSpec((blk, D), lambda i: (i, 0)),
        out_shape=jax.ShapeDtypeStruct((OUT_ROWS, D), jnp.float32),
    )(partials)
    return merged


def kernel(x, e_feat, e_attr, emb_table):
    src = e_feat[0].astype(jnp.int32)
    dst = e_feat[1].astype(jnp.int32)
    code = (e_attr[:, 0] * 32 + e_attr[:, 1]).astype(jnp.int32)
    loop = jnp.arange(N, dtype=jnp.int32)
    pad = IP - 2 * (E + N)
    gl = jnp.concatenate(
        [src, loop, code + N, jnp.full((N,), N + 4 * 32, jnp.int32),
         jnp.zeros((pad,), jnp.int32)]
    )
    dl = jnp.concatenate(
        [dst, loop, dst, loop, jnp.full((pad,), N, jnp.int32)]
    )
    pair = (emb_table[:, None, :] + emb_table[None, :, :]).reshape(1024, D)
    table = jnp.concatenate([x, pair])
    out = _propagate(table, gl, dl)
    return out[:N]
